# Initial kernel scaffold; baseline (speedup 1.0000x reference)
import jax
import jax.numpy as jnp
from jax.experimental import pallas as pl


def kernel(feature, sparse_adj, W, b):
    raise NotImplementedError("write your pallas kernel here")


# speedup vs baseline: 39.5255x; 39.5255x over previous
import functools

import jax
import jax.numpy as jnp
from jax import lax
from jax.experimental import pallas as pl
from jax.experimental.pallas import tpu as pltpu
from jax.experimental.pallas import tpu_sc as plsc

N = 10000
E = 320000
D = 128

NC = 2
NS = 16
NW = NC * NS
EPT = E // NW
B = 50
NB = EPT // B
NBC = 20
NCH = NB // NBC
NP = 10240
NPT = NP // NS
NPR = NP // NS
ZB = 16

_mesh = plsc.VectorSubcoreMesh(core_axis_name="c", subcore_axis_name="s")
_sc_params = pltpu.CompilerParams(needs_layout_passes=False)


@functools.partial(
    pl.kernel,
    out_type=jax.ShapeDtypeStruct((NC, NP), jnp.float32),
    mesh=_mesh,
    scratch_types=[
        pltpu.VMEM((EPT,), jnp.int32),
        pltpu.VMEM((NP,), jnp.float32),
        pltpu.VMEM((NS, NPT), jnp.float32),
        pltpu.VMEM_SHARED((NS, NP), jnp.float32),
    ],
    compiler_params=_sc_params,
)
def _deg_kernel(dst_hbm, pdeg_hbm, dst_v, hist, comb, shared):
    c = lax.axis_index("c")
    s = lax.axis_index("s")
    wid = c * NS + s

    pltpu.sync_copy(dst_hbm.at[wid], dst_v)

    zeros = jnp.zeros((16,), jnp.float32)
    ones = jnp.ones((16,), jnp.float32)

    def _zero(i, _):
        hist[pl.ds(i * 16, 16)] = zeros
        return 0

    lax.fori_loop(0, NP // 16, _zero, 0)

    def _scat(i, _):
        idx = dst_v[pl.ds(i * 16, 16)]
        plsc.addupdate_scatter(hist, [idx], ones)
        return 0

    lax.fori_loop(0, EPT // 16, _scat, 0)

    pltpu.sync_copy(hist, shared.at[s])
    plsc.subcore_barrier()
    pltpu.sync_copy(shared.at[:, pl.ds(s * NPT, NPT)], comb)

    def _red(i, _):
        v = comb[0, pl.ds(i * 16, 16)]
        for r in range(1, NS):
            v = v + comb[r, pl.ds(i * 16, 16)]
        comb[0, pl.ds(i * 16, 16)] = v
        return 0

    lax.fori_loop(0, NPT // 16, _red, 0)
    pltpu.sync_copy(comb.at[0], pdeg_hbm.at[c, pl.ds(s * NPT, NPT)])


def _lin_body(x_ref, w_ref, pd_ref, g_ref):
    h = lax.dot_general(
        x_ref[...], w_ref[...], (((1,), (1,)), ((), ())),
        preferred_element_type=jnp.float32,
    )
    deg = 1.0 + pd_ref[0] + pd_ref[1]
    dis = lax.rsqrt(deg)[:N].reshape(N, 1)
    g_ref[...] = h * dis


_lin_kernel = pl.pallas_call(
    _lin_body, out_shape=jax.ShapeDtypeStruct((N, D), jnp.float32)
)


@functools.partial(
    pl.kernel,
    out_type=jax.ShapeDtypeStruct((NC, NP, D), jnp.float32),
    mesh=_mesh,
    scratch_types=[
        pltpu.VMEM((2, NBC, B), jnp.int32),
        pltpu.VMEM((2, NBC, B), jnp.int32),
        pltpu.VMEM((2, B, D), jnp.float32),
        pltpu.VMEM((ZB, D), jnp.float32),
        pltpu.VMEM_SHARED((NP, D), jnp.float32),
        pltpu.SemaphoreType.DMA,
        pltpu.SemaphoreType.DMA,
        pltpu.SemaphoreType.DMA,
    ],
    compiler_params=_sc_params,
)
def _agg_kernel(
    src_hbm, dst_hbm, g_hbm, acc_hbm, srcd, dstd, buf, zbuf, acc, s0, s1, s2
):
    c = lax.axis_index("c")
    s = lax.axis_index("s")
    wid = c * NS + s

    zeros = jnp.zeros((16,), jnp.float32)

    def _zrow(i, _):
        for j in range(D // 16):
            zbuf[i, pl.ds(j * 16, 16)] = zeros
        return 0

    lax.fori_loop(0, ZB, _zrow, 0)
    for k in range(NPR // ZB):
        pltpu.sync_copy(zbuf, acc.at[pl.ds(s * NPR + k * ZB, ZB)])

    pltpu.sync_copy(src_hbm.at[wid, 0], srcd.at[0])
    pltpu.sync_copy(dst_hbm.at[wid, 0], dstd.at[0])
    plsc.subcore_barrier()

    sems = (s0, s1)

    def _start(ci, blk, slot):
        pltpu.make_async_copy(
            g_hbm.at[srcd.at[ci, blk]], buf.at[slot], sems[slot]
        ).start()

    def _finish(ci, blk, slot):
        pltpu.make_async_copy(
            g_hbm.at[srcd.at[ci, blk]], buf.at[slot], sems[slot]
        ).wait()
        pltpu.sync_copy(buf.at[slot], acc.at[dstd.at[ci, blk]], add=True)

    _start(0, 0, 0)
    for ch in range(NCH):
        ci = ch % 2
        ni = 1 - ci
        if ch + 1 < NCH:
            pltpu.make_async_copy(src_hbm.at[wid, ch + 1], srcd.at[ni], s2).start()
            pltpu.make_async_copy(dst_hbm.at[wid, ch + 1], dstd.at[ni], s2).start()

        def _pair(i, _, ci=ci):
            b0 = i * 2
            _start(ci, b0 + 1, 1)
            _finish(ci, b0, 0)
            _start(ci, b0 + 2, 0)
            _finish(ci, b0 + 1, 1)
            return 0

        lax.fori_loop(0, NBC // 2 - 1, _pair, 0)
        _start(ci, NBC - 1, 1)
        _finish(ci, NBC - 2, 0)
        if ch + 1 < NCH:
            pltpu.make_async_copy(src_hbm.at[wid, ch + 1], srcd.at[ni], s2).wait()
            pltpu.make_async_copy(dst_hbm.at[wid, ch + 1], dstd.at[ni], s2).wait()
            _start(ni, 0, 0)
        _finish(ci, NBC - 1, 1)

    plsc.subcore_barrier()
    pltpu.sync_copy(acc.at[pl.ds(s * NPR, NPR)], acc_hbm.at[c, pl.ds(s * NPR, NPR)])


def _fin_body(acc_ref, g_ref, pd_ref, b_ref, o_ref):
    deg = 1.0 + pd_ref[0] + pd_ref[1]
    dis = lax.rsqrt(deg)[:N].reshape(N, 1)
    total = acc_ref[0, :N, :] + acc_ref[1, :N, :] + g_ref[...]
    o_ref[...] = jnp.maximum(total * dis + b_ref[...][None, :], 0.0)


_fin_kernel = pl.pallas_call(
    _fin_body, out_shape=jax.ShapeDtypeStruct((N, D), jnp.float32)
)


@jax.jit
def kernel(feature, sparse_adj, W, b):
    src3 = sparse_adj[0].reshape(NW, NCH, NBC, B)
    dst3 = sparse_adj[1].reshape(NW, NCH, NBC, B)
    dst2 = sparse_adj[1].reshape(NW, EPT)

    pdeg = _deg_kernel(dst2)
    g = _lin_kernel(feature, W, pdeg)
    acc = _agg_kernel(src3, dst3, g)
    return _fin_kernel(acc, g, pdeg, b)


# speedup vs baseline: 41.9936x; 1.0624x over previous
import functools

import jax
import jax.numpy as jnp
from jax import lax
from jax.experimental import pallas as pl
from jax.experimental.pallas import tpu as pltpu
from jax.experimental.pallas import tpu_sc as plsc

N = 10000
E = 320000
D = 128

NC = 2
NS = 16
NW = NC * NS
EPT = E // NW
B = 50
NB = EPT // B
NBC = 20
NCH = NB // NBC
NP = 10240
NPT = NP // NS
NPR = NP // NS
ZB = 16

_mesh = plsc.VectorSubcoreMesh(core_axis_name="c", subcore_axis_name="s")
_sc_params = pltpu.CompilerParams(needs_layout_passes=False)


@functools.partial(
    pl.kernel,
    out_type=jax.ShapeDtypeStruct((NC, NP), jnp.float32),
    mesh=_mesh,
    scratch_types=[
        pltpu.VMEM((EPT,), jnp.int32),
        pltpu.VMEM((NP,), jnp.float32),
        pltpu.VMEM((NS, NPT), jnp.float32),
        pltpu.VMEM_SHARED((NS, NP), jnp.float32),
    ],
    compiler_params=_sc_params,
)
def _deg_kernel(dst_hbm, pdeg_hbm, dst_v, hist, comb, shared):
    c = lax.axis_index("c")
    s = lax.axis_index("s")
    wid = c * NS + s

    pltpu.sync_copy(dst_hbm.at[wid], dst_v)

    zeros = jnp.zeros((16,), jnp.float32)
    ones = jnp.ones((16,), jnp.float32)

    def _zero(i, _):
        hist[pl.ds(i * 16, 16)] = zeros
        return 0

    lax.fori_loop(0, NP // 16, _zero, 0)

    def _scat(i, _):
        idx = dst_v[pl.ds(i * 16, 16)]
        plsc.addupdate_scatter(hist, [idx], ones)
        return 0

    lax.fori_loop(0, EPT // 16, _scat, 0)

    pltpu.sync_copy(hist, shared.at[s])
    plsc.subcore_barrier()
    pltpu.sync_copy(shared.at[:, pl.ds(s * NPT, NPT)], comb)

    def _red(i, _):
        v = comb[0, pl.ds(i * 16, 16)]
        for r in range(1, NS):
            v = v + comb[r, pl.ds(i * 16, 16)]
        comb[0, pl.ds(i * 16, 16)] = v
        return 0

    lax.fori_loop(0, NPT // 16, _red, 0)
    pltpu.sync_copy(comb.at[0], pdeg_hbm.at[c, pl.ds(s * NPT, NPT)])


def _lin_body(x_ref, w_ref, pd_ref, g_ref):
    h = lax.dot_general(
        x_ref[...], w_ref[...], (((1,), (1,)), ((), ())),
        preferred_element_type=jnp.float32,
    )
    deg = 1.0 + pd_ref[0] + pd_ref[1]
    dis = lax.rsqrt(deg)[:N].reshape(N, 1)
    g_ref[...] = h * dis


_lin_kernel = pl.pallas_call(
    _lin_body, out_shape=jax.ShapeDtypeStruct((N, D), jnp.float32)
)


@functools.partial(
    pl.kernel,
    out_type=jax.ShapeDtypeStruct((NC, NP, D), jnp.float32),
    mesh=_mesh,
    scratch_types=[
        pltpu.VMEM((2, NBC, B), jnp.int32),
        pltpu.VMEM((2, NBC, B), jnp.int32),
        pltpu.VMEM((4, B, D), jnp.float32),
        pltpu.VMEM((ZB, D), jnp.float32),
        pltpu.VMEM_SHARED((NP, D), jnp.float32),
        [pltpu.SemaphoreType.DMA] * 4,
        [pltpu.SemaphoreType.DMA] * 4,
        pltpu.SemaphoreType.DMA,
    ],
    compiler_params=_sc_params,
)
def _agg_kernel(
    src_hbm, dst_hbm, g_hbm, acc_hbm, srcd, dstd, buf, zbuf, acc, gs, ss, s2
):
    c = lax.axis_index("c")
    s = lax.axis_index("s")
    wid = c * NS + s

    zeros = jnp.zeros((16,), jnp.float32)

    def _zrow(i, _):
        for j in range(D // 16):
            zbuf[i, pl.ds(j * 16, 16)] = zeros
        return 0

    lax.fori_loop(0, ZB, _zrow, 0)
    for k in range(NPR // ZB):
        pltpu.sync_copy(zbuf, acc.at[pl.ds(s * NPR + k * ZB, ZB)])

    pltpu.sync_copy(src_hbm.at[wid, 0], srcd.at[0])
    pltpu.sync_copy(dst_hbm.at[wid, 0], dstd.at[0])
    plsc.subcore_barrier()

    def _gath(ci, blk, slot):
        pltpu.make_async_copy(
            g_hbm.at[srcd.at[ci, blk]], buf.at[slot], gs[slot]
        ).start()

    def _wait_g(slot):
        pltpu.make_async_copy(
            g_hbm.at[srcd.at[0, 0]], buf.at[slot], gs[slot]
        ).wait()

    def _scat(ci, blk, slot):
        pltpu.async_copy(
            buf.at[slot], acc.at[dstd.at[ci, blk]], ss[slot], add=True
        )

    def _wait_s(slot):
        pltpu.make_async_copy(
            buf.at[slot], acc.at[dstd.at[0, 0]], ss[slot]
        ).wait()

    _gath(0, 0, 0)
    _gath(0, 1, 1)
    for ch in range(NCH):
        ci = ch % 2
        ni = 1 - ci
        if ch + 1 < NCH:
            pltpu.make_async_copy(src_hbm.at[wid, ch + 1], srcd.at[ni], s2).start()
            pltpu.make_async_copy(dst_hbm.at[wid, ch + 1], dstd.at[ni], s2).start()

        def _step(lb, j, ci=ci):
            _wait_g(j)
            _scat(ci, lb, j)
            _wait_s((j + 2) % 4)
            _gath(ci, lb + 2, (j + 2) % 4)

        if ch == 0:
            _wait_g(0); _scat(0, 0, 0); _gath(0, 2, 2)
            _wait_g(1); _scat(0, 1, 1); _gath(0, 3, 3)
            _step(2, 2)
            _step(3, 3)
            q_lo = 1
        else:
            q_lo = 0

        def _quad(q, _, ci=ci):
            lb = q * 4
            for j in range(4):
                _step(lb + j, j, ci=ci)
            return 0

        lax.fori_loop(q_lo, NBC // 4 - 1, _quad, 0)

        _step(NBC - 4, 0)
        _step(NBC - 3, 1)
        if ch + 1 < NCH:
            pltpu.make_async_copy(src_hbm.at[wid, ch + 1], srcd.at[ni], s2).wait()
            pltpu.make_async_copy(dst_hbm.at[wid, ch + 1], dstd.at[ni], s2).wait()
        _wait_g(2); _scat(ci, NBC - 2, 2)
        _wait_g(3); _scat(ci, NBC - 1, 3)
        if ch + 1 < NCH:
            _wait_s(0); _gath(ni, 0, 0)
            _wait_s(1); _gath(ni, 1, 1)

    _wait_s(0)
    _wait_s(1)
    _wait_s(2)
    _wait_s(3)
    plsc.subcore_barrier()
    pltpu.sync_copy(acc.at[pl.ds(s * NPR, NPR)], acc_hbm.at[c, pl.ds(s * NPR, NPR)])


def _fin_body(acc_ref, g_ref, pd_ref, b_ref, o_ref):
    deg = 1.0 + pd_ref[0] + pd_ref[1]
    dis = lax.rsqrt(deg)[:N].reshape(N, 1)
    total = acc_ref[0, :N, :] + acc_ref[1, :N, :] + g_ref[...]
    o_ref[...] = jnp.maximum(total * dis + b_ref[...][None, :], 0.0)


_fin_kernel = pl.pallas_call(
    _fin_body, out_shape=jax.ShapeDtypeStruct((N, D), jnp.float32)
)


@jax.jit
def kernel(feature, sparse_adj, W, b):
    src3 = sparse_adj[0].reshape(NW, NCH, NBC, B)
    dst3 = sparse_adj[1].reshape(NW, NCH, NBC, B)
    dst2 = sparse_adj[1].reshape(NW, EPT)

    pdeg = _deg_kernel(dst2)
    g = _lin_kernel(feature, W, pdeg)
    acc = _agg_kernel(src3, dst3, g)
    return _fin_kernel(acc, g, pdeg, b)


# speedup vs baseline: 42.5756x; 1.0139x over previous
import functools

import jax
import jax.numpy as jnp
from jax import lax
from jax.experimental import pallas as pl
from jax.experimental.pallas import tpu as pltpu
from jax.experimental.pallas import tpu_sc as plsc

N = 10000
E = 320000
D = 128

NC = 2
NS = 16
NW = NC * NS
EPT = E // NW
B = 125
NB = EPT // B
NBC = 20
NCH = NB // NBC
NP = 10240
NPT = NP // NS
NPR = NP // NS
ZB = 16

_mesh = plsc.VectorSubcoreMesh(core_axis_name="c", subcore_axis_name="s")
_sc_params = pltpu.CompilerParams(needs_layout_passes=False)


@functools.partial(
    pl.kernel,
    out_type=jax.ShapeDtypeStruct((NC, NP), jnp.float32),
    mesh=_mesh,
    scratch_types=[
        pltpu.VMEM((EPT,), jnp.int32),
        pltpu.VMEM((NP,), jnp.float32),
        pltpu.VMEM((NS, NPT), jnp.float32),
        pltpu.VMEM_SHARED((NS, NP), jnp.float32),
    ],
    compiler_params=_sc_params,
)
def _deg_kernel(dst_hbm, pdeg_hbm, dst_v, hist, comb, shared):
    c = lax.axis_index("c")
    s = lax.axis_index("s")
    wid = c * NS + s

    pltpu.sync_copy(dst_hbm.at[wid], dst_v)

    zeros = jnp.zeros((16,), jnp.float32)
    ones = jnp.ones((16,), jnp.float32)

    def _zero(i, _):
        hist[pl.ds(i * 16, 16)] = zeros
        return 0

    lax.fori_loop(0, NP // 16, _zero, 0)

    def _scat(i, _):
        idx = dst_v[pl.ds(i * 16, 16)]
        plsc.addupdate_scatter(hist, [idx], ones)
        return 0

    lax.fori_loop(0, EPT // 16, _scat, 0)

    pltpu.sync_copy(hist, shared.at[s])
    plsc.subcore_barrier()
    pltpu.sync_copy(shared.at[:, pl.ds(s * NPT, NPT)], comb)

    def _red(i, _):
        v = comb[0, pl.ds(i * 16, 16)]
        for r in range(1, NS):
            v = v + comb[r, pl.ds(i * 16, 16)]
        comb[0, pl.ds(i * 16, 16)] = v
        return 0

    lax.fori_loop(0, NPT // 16, _red, 0)
    pltpu.sync_copy(comb.at[0], pdeg_hbm.at[c, pl.ds(s * NPT, NPT)])


def _lin_body(x_ref, w_ref, pd_ref, g_ref):
    h = lax.dot_general(
        x_ref[...], w_ref[...], (((1,), (1,)), ((), ())),
        preferred_element_type=jnp.float32,
    )
    deg = 1.0 + pd_ref[0] + pd_ref[1]
    dis = lax.rsqrt(deg)[:N].reshape(N, 1)
    g_ref[...] = h * dis


_lin_kernel = pl.pallas_call(
    _lin_body, out_shape=jax.ShapeDtypeStruct((N, D), jnp.float32)
)


@functools.partial(
    pl.kernel,
    out_type=jax.ShapeDtypeStruct((NC, NP, D), jnp.float32),
    mesh=_mesh,
    scratch_types=[
        pltpu.VMEM((2, NBC, B), jnp.int32),
        pltpu.VMEM((2, NBC, B), jnp.int32),
        pltpu.VMEM((2, B, D), jnp.float32),
        pltpu.VMEM((ZB, D), jnp.float32),
        pltpu.VMEM_SHARED((NP, D), jnp.float32),
        [pltpu.SemaphoreType.DMA] * 2,
        [pltpu.SemaphoreType.DMA] * 2,
        pltpu.SemaphoreType.DMA,
    ],
    compiler_params=_sc_params,
)
def _agg_kernel(
    src_hbm, dst_hbm, g_hbm, acc_hbm, srcd, dstd, buf, zbuf, acc, gs, ss, s2
):
    c = lax.axis_index("c")
    s = lax.axis_index("s")
    wid = c * NS + s

    zeros = jnp.zeros((16,), jnp.float32)

    def _zrow(i, _):
        for j in range(D // 16):
            zbuf[i, pl.ds(j * 16, 16)] = zeros
        return 0

    lax.fori_loop(0, ZB, _zrow, 0)
    for k in range(NPR // ZB):
        pltpu.sync_copy(zbuf, acc.at[pl.ds(s * NPR + k * ZB, ZB)])

    pltpu.sync_copy(src_hbm.at[wid, 0], srcd.at[0])
    pltpu.sync_copy(dst_hbm.at[wid, 0], dstd.at[0])
    plsc.subcore_barrier()

    def _gath(ci, blk, slot):
        pltpu.make_async_copy(
            g_hbm.at[srcd.at[ci, blk]], buf.at[slot], gs[slot]
        ).start()

    def _wait_g(slot):
        pltpu.make_async_copy(
            g_hbm.at[srcd.at[0, 0]], buf.at[slot], gs[slot]
        ).wait()

    def _scat(ci, blk, slot):
        pltpu.async_copy(
            buf.at[slot], acc.at[dstd.at[ci, blk]], ss[slot], add=True
        )

    def _wait_s(slot):
        pltpu.make_async_copy(
            buf.at[slot], acc.at[dstd.at[0, 0]], ss[slot]
        ).wait()

    _gath(0, 0, 0)
    for ch in range(NCH):
        ci = ch % 2
        ni = 1 - ci
        if ch + 1 < NCH:
            pltpu.make_async_copy(src_hbm.at[wid, ch + 1], srcd.at[ni], s2).start()
            pltpu.make_async_copy(dst_hbm.at[wid, ch + 1], dstd.at[ni], s2).start()

        def _step(lb, j, ci=ci):
            _wait_g(j)
            _scat(ci, lb, j)
            _wait_s(1 - j)
            _gath(ci, lb + 1, 1 - j)

        if ch == 0:
            _wait_g(0); _scat(0, 0, 0); _gath(0, 1, 1)
            _step(1, 1)
            q_lo = 1
        else:
            q_lo = 0

        def _pair(q, _, ci=ci):
            _step(q * 2, 0, ci=ci)
            _step(q * 2 + 1, 1, ci=ci)
            return 0

        lax.fori_loop(q_lo, NBC // 2 - 1, _pair, 0)

        _step(NBC - 2, 0)
        if ch + 1 < NCH:
            pltpu.make_async_copy(src_hbm.at[wid, ch + 1], srcd.at[ni], s2).wait()
            pltpu.make_async_copy(dst_hbm.at[wid, ch + 1], dstd.at[ni], s2).wait()
            _wait_g(1); _scat(ci, NBC - 1, 1)
            _wait_s(0); _gath(ni, 0, 0)
        else:
            _wait_g(1); _scat(ci, NBC - 1, 1)

    _wait_s(0)
    _wait_s(1)
    plsc.subcore_barrier()
    pltpu.sync_copy(acc.at[pl.ds(s * NPR, NPR)], acc_hbm.at[c, pl.ds(s * NPR, NPR)])


def _fin_body(acc_ref, g_ref, pd_ref, b_ref, o_ref):
    deg = 1.0 + pd_ref[0] + pd_ref[1]
    dis = lax.rsqrt(deg)[:N].reshape(N, 1)
    total = acc_ref[0, :N, :] + acc_ref[1, :N, :] + g_ref[...]
    o_ref[...] = jnp.maximum(total * dis + b_ref[...][None, :], 0.0)


_fin_kernel = pl.pallas_call(
    _fin_body, out_shape=jax.ShapeDtypeStruct((N, D), jnp.float32)
)


@jax.jit
def kernel(feature, sparse_adj, W, b):
    src3 = sparse_adj[0].reshape(NW, NCH, NBC, B)
    dst3 = sparse_adj[1].reshape(NW, NCH, NBC, B)
    dst2 = sparse_adj[1].reshape(NW, EPT)

    pdeg = _deg_kernel(dst2)
    g = _lin_kernel(feature, W, pdeg)
    acc = _agg_kernel(src3, dst3, g)
    return _fin_kernel(acc, g, pdeg, b)


# speedup vs baseline: 50.2109x; 1.1793x over previous
import functools

import jax
import jax.numpy as jnp
from jax import lax
from jax.experimental import pallas as pl
from jax.experimental.pallas import tpu as pltpu
from jax.experimental.pallas import tpu_sc as plsc

N = 10000
E = 320000
D = 128

NC = 2
NS = 16
NW = NC * NS
EPT = E // NW
B = 50
NB = EPT // B
NBC = 20
NCH = NB // NBC
K = 5
NP = 10240
NPT = NP // NS
NPA = 10112
NPR = NPA // NS
ZB = 8

_mesh = plsc.VectorSubcoreMesh(core_axis_name="c", subcore_axis_name="s")
_sc_params = pltpu.CompilerParams(needs_layout_passes=False)


@functools.partial(
    pl.kernel,
    out_type=jax.ShapeDtypeStruct((NC, NP), jnp.float32),
    mesh=_mesh,
    scratch_types=[
        pltpu.VMEM((EPT,), jnp.int32),
        pltpu.VMEM((NP,), jnp.float32),
        pltpu.VMEM((NS, NPT), jnp.float32),
        pltpu.VMEM_SHARED((NS, NP), jnp.float32),
    ],
    compiler_params=_sc_params,
)
def _deg_kernel(dst_hbm, pdeg_hbm, dst_v, hist, comb, shared):
    c = lax.axis_index("c")
    s = lax.axis_index("s")
    wid = c * NS + s

    pltpu.sync_copy(dst_hbm.at[wid], dst_v)

    zeros = jnp.zeros((16,), jnp.float32)
    ones = jnp.ones((16,), jnp.float32)

    def _zero(i, _):
        hist[pl.ds(i * 16, 16)] = zeros
        return 0

    lax.fori_loop(0, NP // 16, _zero, 0)

    def _scat(i, _):
        idx = dst_v[pl.ds(i * 16, 16)]
        plsc.addupdate_scatter(hist, [idx], ones)
        return 0

    lax.fori_loop(0, EPT // 16, _scat, 0)

    pltpu.sync_copy(hist, shared.at[s])
    plsc.subcore_barrier()
    pltpu.sync_copy(shared.at[:, pl.ds(s * NPT, NPT)], comb)

    def _red(i, _):
        v = comb[0, pl.ds(i * 16, 16)]
        for r in range(1, NS):
            v = v + comb[r, pl.ds(i * 16, 16)]
        comb[0, pl.ds(i * 16, 16)] = v
        return 0

    lax.fori_loop(0, NPT // 16, _red, 0)
    pltpu.sync_copy(comb.at[0], pdeg_hbm.at[c, pl.ds(s * NPT, NPT)])


def _lin_body(x_ref, w_ref, pd_ref, g_ref):
    h = lax.dot_general(
        x_ref[...], w_ref[...], (((1,), (1,)), ((), ())),
        preferred_element_type=jnp.float32,
    )
    deg = 1.0 + pd_ref[0] + pd_ref[1]
    dis = lax.rsqrt(deg)[:N].reshape(N, 1)
    g_ref[...] = h * dis


_lin_kernel = pl.pallas_call(
    _lin_body, out_shape=jax.ShapeDtypeStruct((N, D), jnp.float32)
)


@functools.partial(
    pl.kernel,
    out_type=jax.ShapeDtypeStruct((NC, NPA, D), jnp.float32),
    mesh=_mesh,
    scratch_types=[
        pltpu.VMEM((2, NBC, B), jnp.int32),
        pltpu.VMEM((2, NBC, B), jnp.int32),
        pltpu.VMEM((K, B, D), jnp.float32),
        pltpu.VMEM((ZB, D), jnp.float32),
        pltpu.VMEM_SHARED((NPA, D), jnp.float32),
        [pltpu.SemaphoreType.DMA] * K,
        [pltpu.SemaphoreType.DMA] * K,
        pltpu.SemaphoreType.DMA,
    ],
    compiler_params=_sc_params,
)
def _agg_kernel(
    src_hbm, dst_hbm, g_hbm, acc_hbm, srcd, dstd, buf, zbuf, acc, gs, ss, s2
):
    c = lax.axis_index("c")
    s = lax.axis_index("s")
    wid = c * NS + s

    zeros = jnp.zeros((16,), jnp.float32)

    def _zrow(i, _):
        for j in range(D // 16):
            zbuf[i, pl.ds(j * 16, 16)] = zeros
        return 0

    lax.fori_loop(0, ZB, _zrow, 0)
    for k in range(NPR // ZB):
        pltpu.sync_copy(zbuf, acc.at[pl.ds(s * NPR + k * ZB, ZB)])

    pltpu.sync_copy(src_hbm.at[wid, 0], srcd.at[0])
    pltpu.sync_copy(dst_hbm.at[wid, 0], dstd.at[0])
    plsc.subcore_barrier()

    def _gath(ci, blk, slot):
        pltpu.make_async_copy(
            g_hbm.at[srcd.at[ci, blk]], buf.at[slot], gs[slot]
        ).start()

    def _wait_g(slot):
        pltpu.make_async_copy(
            g_hbm.at[srcd.at[0, 0]], buf.at[slot], gs[slot]
        ).wait()

    def _scat(ci, blk, slot):
        pltpu.async_copy(
            buf.at[slot], acc.at[dstd.at[ci, blk]], ss[slot], add=True
        )

    def _wait_s(slot):
        pltpu.make_async_copy(
            buf.at[slot], acc.at[dstd.at[0, 0]], ss[slot]
        ).wait()

    for j in range(K - 1):
        _gath(0, j, j)
    for ch in range(NCH):
        ci = ch % 2
        ni = 1 - ci
        if ch + 1 < NCH:
            pltpu.make_async_copy(src_hbm.at[wid, ch + 1], srcd.at[ni], s2).start()
            pltpu.make_async_copy(dst_hbm.at[wid, ch + 1], dstd.at[ni], s2).start()

        def _step(lb, j, ci=ci, nci=None):
            _wait_g(j)
            _scat(ci, lb, j)
            jl = (j + K - 1) % K
            _wait_s(jl)
            if nci is None:
                _gath(ci, lb + K - 1, jl)
            else:
                _gath(nci, (lb + K - 1) % NBC, jl)

        if ch == 0:
            _wait_g(0); _scat(0, 0, 0); _gath(0, K - 1, K - 1)
            for j in range(1, K):
                _step(j, j)
            c_lo = 1
        else:
            c_lo = 0

        def _cycle(q, _, ci=ci):
            lb = q * K
            for j in range(K):
                _step(lb + j, j, ci=ci)
            return 0

        lax.fori_loop(c_lo, NBC // K - 1, _cycle, 0)

        lb0 = NBC - K
        _step(lb0, 0)
        if ch + 1 < NCH:
            pltpu.make_async_copy(src_hbm.at[wid, ch + 1], srcd.at[ni], s2).wait()
            pltpu.make_async_copy(dst_hbm.at[wid, ch + 1], dstd.at[ni], s2).wait()
            for j in range(1, K):
                _step(lb0 + j, j, nci=ni)
        else:
            for j in range(1, K):
                _wait_g(j)
                _scat(ci, lb0 + j, j)

    for j in range(K):
        _wait_s(j)
    plsc.subcore_barrier()
    pltpu.sync_copy(acc.at[pl.ds(s * NPR, NPR)], acc_hbm.at[c, pl.ds(s * NPR, NPR)])


def _fin_body(acc_ref, g_ref, pd_ref, b_ref, o_ref):
    deg = 1.0 + pd_ref[0] + pd_ref[1]
    dis = lax.rsqrt(deg)[:N].reshape(N, 1)
    total = acc_ref[0, :N, :] + acc_ref[1, :N, :] + g_ref[...]
    o_ref[...] = jnp.maximum(total * dis + b_ref[...][None, :], 0.0)


_fin_kernel = pl.pallas_call(
    _fin_body, out_shape=jax.ShapeDtypeStruct((N, D), jnp.float32)
)


@jax.jit
def kernel(feature, sparse_adj, W, b):
    src3 = sparse_adj[0].reshape(NW, NCH, NBC, B)
    dst3 = sparse_adj[1].reshape(NW, NCH, NBC, B)
    dst2 = sparse_adj[1].reshape(NW, EPT)

    pdeg = _deg_kernel(dst2)
    g = _lin_kernel(feature, W, pdeg)
    acc = _agg_kernel(src3, dst3, g)
    return _fin_kernel(acc, g, pdeg, b)
